# baseline (speedup 1.0000x reference)
import functools

import jax
import jax.numpy as jnp
from jax import lax
from jax.experimental import pallas as pl
from jax.experimental.pallas import tpu as pltpu

_EPS = 1e-12


def _addnorm_body(x_ref, y_ref, gamma_ref, beta_ref, o_ref, *, inv_h):
    s = x_ref[...].astype(jnp.float32) + y_ref[...].astype(jnp.float32)
    ssum = jnp.sum(s, axis=-1, keepdims=True)
    s2sum = jnp.sum(s * s, axis=-1, keepdims=True)
    mean = ssum * inv_h
    var = s2sum * inv_h - mean * mean
    inv_std = lax.rsqrt(var + _EPS)
    o_ref[...] = ((s - mean) * inv_std * gamma_ref[...] + beta_ref[...]).astype(o_ref.dtype)


def _row_tile(rows, width, itemsize, budget_bytes=30 << 20):
    sub = 8 if itemsize >= 4 else (16 if itemsize == 2 else 32)
    bytes_per_row = width * (3 * 2 * itemsize + 3 * 4)
    cap = max(sub, int(budget_bytes // max(1, bytes_per_row)))
    cap = (cap // sub) * sub
    if rows <= cap:
        if rows <= sub:
            return rows
        half = -(-rows // 2)
        return min(-(-half // sub) * sub, rows)
    return cap


def kernel(x, y, gamma, beta):
    assert x.shape == y.shape
    H = x.shape[-1]
    orig_shape = x.shape
    out_dtype = x.dtype
    itemsize = jnp.dtype(x.dtype).itemsize

    x2 = x.reshape(-1, H)
    y2 = y.reshape(-1, H)
    rows = x2.shape[0]

    gamma2 = gamma.reshape(1, H).astype(jnp.float32)
    beta2 = beta.reshape(1, H).astype(jnp.float32)

    tile = 512 if rows % 512 == 0 else _row_tile(rows, H, itemsize)
    grid = (pl.cdiv(rows, tile),)
    body = functools.partial(_addnorm_body, inv_h=1.0 / H)

    out = pl.pallas_call(
        body,
        out_shape=jax.ShapeDtypeStruct((rows, H), out_dtype),
        grid_spec=pl.GridSpec(
            grid=grid,
            in_specs=[
                pl.BlockSpec((tile, H), lambda i: (i, 0)),
                pl.BlockSpec((tile, H), lambda i: (i, 0)),
                pl.BlockSpec((1, H), lambda i: (0, 0)),
                pl.BlockSpec((1, H), lambda i: (0, 0)),
            ],
            out_specs=pl.BlockSpec((tile, H), lambda i: (i, 0)),
        ),
        compiler_params=pltpu.CompilerParams(
            dimension_semantics=("parallel",),
            vmem_limit_bytes=int(min(
                3 * 2 * tile * H * itemsize + 4 * tile * H * 4 + (4 << 20),
                58 << 20)),
        ),
        cost_estimate=pl.CostEstimate(
            flops=int(8 * rows * H),
            transcendentals=int(rows),
            bytes_accessed=int(3 * rows * H * itemsize),
        ),
    )(x2, y2, gamma2, beta2)
    return out.reshape(orig_shape)


# speedup vs baseline: 1.0625x; 1.0625x over previous
import functools

import jax
import jax.numpy as jnp
from jax import lax
from jax.experimental import pallas as pl
from jax.experimental.pallas import tpu as pltpu

_EPS = 1e-12


def _addnorm_body(x_ref, y_ref, gamma_ref, beta_ref, o_ref, *, inv_h):
    s = x_ref[...].astype(jnp.float32) + y_ref[...].astype(jnp.float32)
    ssum = jnp.sum(s, axis=-1, keepdims=True)
    s2sum = jnp.sum(s * s, axis=-1, keepdims=True)
    mean = ssum * inv_h
    var = s2sum * inv_h - mean * mean
    inv_std = lax.rsqrt(var + _EPS)
    o_ref[...] = ((s - mean) * inv_std * gamma_ref[...] + beta_ref[...]).astype(o_ref.dtype)


def _row_tile(rows, width, itemsize, budget_bytes=30 << 20):
    sub = 8 if itemsize >= 4 else (16 if itemsize == 2 else 32)
    bytes_per_row = width * (3 * 2 * itemsize + 3 * 4)
    cap = max(sub, int(budget_bytes // max(1, bytes_per_row)))
    cap = (cap // sub) * sub
    if rows <= cap:
        if rows <= sub:
            return rows
        half = -(-rows // 2)
        return min(-(-half // sub) * sub, rows)
    return cap


def kernel(x, y, gamma, beta):
    assert x.shape == y.shape
    H = x.shape[-1]
    orig_shape = x.shape
    out_dtype = x.dtype
    itemsize = jnp.dtype(x.dtype).itemsize

    x2 = x.reshape(-1, H)
    y2 = y.reshape(-1, H)
    rows = x2.shape[0]

    gamma2 = gamma.reshape(1, H).astype(jnp.float32)
    beta2 = beta.reshape(1, H).astype(jnp.float32)

    tile = 2048 if rows % 2048 == 0 else _row_tile(rows, H, itemsize)
    grid = (pl.cdiv(rows, tile),)
    body = functools.partial(_addnorm_body, inv_h=1.0 / H)

    out = pl.pallas_call(
        body,
        out_shape=jax.ShapeDtypeStruct((rows, H), out_dtype),
        grid_spec=pl.GridSpec(
            grid=grid,
            in_specs=[
                pl.BlockSpec((tile, H), lambda i: (i, 0)),
                pl.BlockSpec((tile, H), lambda i: (i, 0)),
                pl.BlockSpec((1, H), lambda i: (0, 0)),
                pl.BlockSpec((1, H), lambda i: (0, 0)),
            ],
            out_specs=pl.BlockSpec((tile, H), lambda i: (i, 0)),
        ),
        compiler_params=pltpu.CompilerParams(
            dimension_semantics=("parallel",),
            vmem_limit_bytes=int(min(
                3 * 2 * tile * H * itemsize + 4 * tile * H * 4 + (4 << 20),
                58 << 20)),
        ),
        cost_estimate=pl.CostEstimate(
            flops=int(8 * rows * H),
            transcendentals=int(rows),
            bytes_accessed=int(3 * rows * H * itemsize),
        ),
    )(x2, y2, gamma2, beta2)
    return out.reshape(orig_shape)


# speedup vs baseline: 1.0626x; 1.0001x over previous
import functools

import jax
import jax.numpy as jnp
from jax import lax
from jax.experimental import pallas as pl
from jax.experimental.pallas import tpu as pltpu

_EPS = 1e-12


def _addnorm_body(x_ref, y_ref, gamma_ref, beta_ref, o_ref, *, inv_h):
    s = x_ref[...].astype(jnp.float32) + y_ref[...].astype(jnp.float32)
    ssum = jnp.sum(s, axis=-1, keepdims=True)
    s2sum = jnp.sum(s * s, axis=-1, keepdims=True)
    mean = ssum * inv_h
    var = s2sum * inv_h - mean * mean
    inv_std = lax.rsqrt(var + _EPS)
    o_ref[...] = ((s - mean) * inv_std * gamma_ref[...] + beta_ref[...]).astype(o_ref.dtype)


def _row_tile(rows, width, itemsize, budget_bytes=30 << 20):
    sub = 8 if itemsize >= 4 else (16 if itemsize == 2 else 32)
    bytes_per_row = width * (3 * 2 * itemsize + 3 * 4)
    cap = max(sub, int(budget_bytes // max(1, bytes_per_row)))
    cap = (cap // sub) * sub
    if rows <= cap:
        if rows <= sub:
            return rows
        half = -(-rows // 2)
        return min(-(-half // sub) * sub, rows)
    return cap


def kernel(x, y, gamma, beta):
    assert x.shape == y.shape
    H = x.shape[-1]
    orig_shape = x.shape
    out_dtype = x.dtype
    itemsize = jnp.dtype(x.dtype).itemsize

    x2 = x.reshape(-1, H)
    y2 = y.reshape(-1, H)
    rows = x2.shape[0]

    gamma2 = gamma.reshape(1, H).astype(jnp.float32)
    beta2 = beta.reshape(1, H).astype(jnp.float32)

    tile = 2048 if rows % 2048 == 0 else _row_tile(rows, H, itemsize)
    nblk = pl.cdiv(rows, tile)
    body = functools.partial(_addnorm_body, inv_h=1.0 / H)

    if nblk % 2 == 0:
        half = nblk // 2
        grid = (2, half)
        sem = ("parallel", "arbitrary")
        data_spec = pl.BlockSpec((tile, H), lambda c, i: (c * half + i, 0))
        vec_spec = pl.BlockSpec((1, H), lambda c, i: (0, 0))
    else:
        grid = (nblk,)
        sem = ("parallel",)
        data_spec = pl.BlockSpec((tile, H), lambda i: (i, 0))
        vec_spec = pl.BlockSpec((1, H), lambda i: (0, 0))

    out = pl.pallas_call(
        body,
        out_shape=jax.ShapeDtypeStruct((rows, H), out_dtype),
        grid_spec=pl.GridSpec(
            grid=grid,
            in_specs=[data_spec, data_spec, vec_spec, vec_spec],
            out_specs=data_spec,
        ),
        compiler_params=pltpu.CompilerParams(
            dimension_semantics=sem,
            vmem_limit_bytes=int(min(
                3 * 2 * tile * H * itemsize + 4 * tile * H * 4 + (4 << 20),
                58 << 20)),
        ),
        cost_estimate=pl.CostEstimate(
            flops=int(8 * rows * H),
            transcendentals=int(rows),
            bytes_accessed=int(3 * rows * H * itemsize),
        ),
    )(x2, y2, gamma2, beta2)
    return out.reshape(orig_shape)
